# baseline (speedup 1.0000x reference)
import jax
import jax.numpy as jnp
from jax import lax
from jax.experimental import pallas as pl
from jax.experimental.pallas import tpu as pltpu
from jax.experimental.pallas import tpu_sc as plsc

N = 10000
E = 320000
B = 16
H = 128

NB_N = 1000
NB_E = 2560
CHUNK = 128
NCHUNK = E // CHUNK
NC, NS = 2, 16
NW = NC * NS
NT = -(-NCHUNK // NW)
NP = 10240
SZ = NP // NS

_f32 = jnp.float32


def _relu(v):
    return jnp.maximum(v, 0.0)


def _prep_body(cond, cw1, cb1, cw2, cb2, wd1, wd2, vc1, vc2,
               uA1, uA2, uC1, uC2):
    u = _relu(jnp.dot(cond[...], cw1[...]) + cb1[...])
    u = jnp.dot(u, cw2[...]) + cb2[...]
    uA1[...] = jnp.dot(u, wd1[...])
    uA2[...] = jnp.dot(u, wd2[...])
    uC1[...] = jnp.dot(u, vc1[...])
    uC2[...] = jnp.dot(u, vc2[...])


def _k1_body(x, bf, w1, b1, w2, b2, wa, wb, ua, b1e,
             xh_o, oh_o, a_o, b_o):
    h = _relu(jnp.dot(x[...], w1[...]) + b1[...])
    xh = jnp.dot(h, w2[...]) + b2[...]
    iot = lax.broadcasted_iota(jnp.int32, (NB_N, B), 1).astype(_f32)
    oh = (bf[...] == iot).astype(_f32)
    xh_o[...] = xh
    oh_o[...] = oh
    a_o[...] = jnp.dot(xh, wa[...]) + jnp.dot(oh, ua[...]) + b1e[...]
    b_o[...] = jnp.dot(xh, wb[...])


def _k2_body(ea, w1, b1, w2, b2, out):
    h = _relu(jnp.dot(ea[...], w1[...]) + b1[...])
    out[...] = jnp.dot(h, w2[...]) + b2[...]


def _k3_body(ehp, ga, gb, w1c, w2, b2, out):
    pre = jnp.dot(ehp[...], w1c[...]) + ga[...] + gb[...]
    out[...] = jnp.dot(_relu(pre), w2[...]) + b2[...]


def _node_update(xh, sums, cnt, oh, va, vb, uc, nb1, nw2, nb2):
    s = sums[0] + sums[1]
    ct = (cnt[0] + cnt[1])[:, 0:1]
    agg = s / jnp.maximum(ct, 1.0)
    h = _relu(jnp.dot(xh, va[...]) + jnp.dot(agg, vb[...])
              + jnp.dot(oh, uc[...]) + nb1[...])
    return jnp.dot(h, nw2[...]) + nb2[...] + xh


def _k4a_body(xh, sums, cnt, oh, va, vb, uc, nb1, nw2, nb2,
              wa2, wb2, ua2, b12, xh2_o, a_o, b_o):
    xh2 = _node_update(xh[...], sums[...], cnt[...], oh[...],
                       va, vb, uc, nb1, nw2, nb2)
    xh2_o[...] = xh2
    a_o[...] = (jnp.dot(xh2, wa2[...]) + jnp.dot(oh[...], ua2[...])
                + b12[...])
    b_o[...] = jnp.dot(xh2, wb2[...])


def _k4b_body(xh, sums, cnt, oh, va, vb, uc, nb1, nw2, nb2,
              dw1, db1, dw2, db2, out):
    xh2 = _node_update(xh[...], sums[...], cnt[...], oh[...],
                       va, vb, uc, nb1, nw2, nb2)
    h = _relu(jnp.dot(xh2, dw1[...]) + db1[...])
    out[...] = jnp.dot(h, dw2[...]) + db2[...]


def _full(shape):
    return pl.BlockSpec(shape, lambda *_: tuple(0 for _ in shape))


def _prep(cond, ce, wd1, wd2, vc1, vc2):
    outs = [jax.ShapeDtypeStruct((B, H), _f32)] * 4
    return pl.pallas_call(
        _prep_body,
        out_shape=outs,
    )(cond, ce['w1'], ce['b1'].reshape(1, H), ce['w2'], ce['b2'].reshape(1, H),
      wd1, wd2, vc1, vc2)


def _k1(x, batchf, ne, wa, wb, ua, b1e):
    nb = N // NB_N
    row_spec = pl.BlockSpec((NB_N, H), lambda i: (i, 0))
    outs = [jax.ShapeDtypeStruct((N, H), _f32),
            jax.ShapeDtypeStruct((N, B), _f32),
            jax.ShapeDtypeStruct((N, H), _f32),
            jax.ShapeDtypeStruct((N, H), _f32)]
    return pl.pallas_call(
        _k1_body,
        grid=(nb,),
        in_specs=[row_spec,
                  pl.BlockSpec((NB_N, 1), lambda i: (i, 0)),
                  _full((H, H)), _full((1, H)), _full((H, H)), _full((1, H)),
                  _full((H, H)), _full((H, H)), _full((B, H)), _full((1, H))],
        out_specs=[row_spec,
                   pl.BlockSpec((NB_N, B), lambda i: (i, 0)),
                   row_spec, row_spec],
        out_shape=outs,
    )(x, batchf, ne['w1'], ne['b1'].reshape(1, H), ne['w2'],
      ne['b2'].reshape(1, H), wa, wb, ua, b1e)


def _k2(ea, ee):
    nb = E // NB_E
    return pl.pallas_call(
        _k2_body,
        grid=(nb,),
        in_specs=[pl.BlockSpec((NB_E, 16), lambda i: (i, 0)),
                  _full((16, H)), _full((1, H)), _full((H, H)), _full((1, H))],
        out_specs=pl.BlockSpec((NB_E, H), lambda i: (i, 0)),
        out_shape=jax.ShapeDtypeStruct((E, H), _f32),
    )(ea, ee['w1'], ee['b1'].reshape(1, H), ee['w2'], ee['b2'].reshape(1, H))


def _k3(ehp, ga, gb, w1c, w2, b2):
    nb = E // NB_E
    row_spec = pl.BlockSpec((NB_E, H), lambda i: (i, 0))
    return pl.pallas_call(
        _k3_body,
        grid=(nb,),
        in_specs=[row_spec, row_spec, row_spec,
                  _full((H, H)), _full((H, H)), _full((1, H))],
        out_specs=row_spec,
        out_shape=jax.ShapeDtypeStruct((E, H), _f32),
    )(ehp, ga, gb, w1c, w2, b2.reshape(1, H))


def _k4a(xh, sums, cnt, oh, nl, uc, wa2, wb2, ua2, b12):
    nb = N // NB_N
    row_spec = pl.BlockSpec((NB_N, H), lambda i: (i, 0))
    outs = [jax.ShapeDtypeStruct((N, H), _f32)] * 3
    return pl.pallas_call(
        _k4a_body,
        grid=(nb,),
        in_specs=[row_spec,
                  pl.BlockSpec((NC, NB_N, H), lambda i: (0, i, 0)),
                  pl.BlockSpec((NC, NB_N, H), lambda i: (0, i, 0)),
                  pl.BlockSpec((NB_N, B), lambda i: (i, 0)),
                  _full((H, H)), _full((H, H)), _full((B, H)), _full((1, H)),
                  _full((H, H)), _full((1, H)),
                  _full((H, H)), _full((H, H)), _full((B, H)), _full((1, H))],
        out_specs=[row_spec, row_spec, row_spec],
        out_shape=outs,
    )(xh, sums, cnt, oh,
      nl['va'], nl['vb'], uc, nl['b1'], nl['w2'], nl['b2'],
      wa2, wb2, ua2, b12)


def _k4b(xh, sums, cnt, oh, nl, uc, dec):
    nb = N // NB_N
    row_spec = pl.BlockSpec((NB_N, H), lambda i: (i, 0))
    return pl.pallas_call(
        _k4b_body,
        grid=(nb,),
        in_specs=[row_spec,
                  pl.BlockSpec((NC, NB_N, H), lambda i: (0, i, 0)),
                  pl.BlockSpec((NC, NB_N, H), lambda i: (0, i, 0)),
                  pl.BlockSpec((NB_N, B), lambda i: (i, 0)),
                  _full((H, H)), _full((H, H)), _full((B, H)), _full((1, H)),
                  _full((H, H)), _full((1, H)),
                  _full((H, H)), _full((1, H)), _full((H, H)), _full((1, H))],
        out_specs=row_spec,
        out_shape=jax.ShapeDtypeStruct((N, H), _f32),
    )(xh, sums, cnt, oh,
      nl['va'], nl['vb'], uc, nl['b1'], nl['w2'], nl['b2'],
      dec['w1'], dec['b1'].reshape(1, H), dec['w2'], dec['b2'].reshape(1, H))


def _mesh():
    return plsc.VectorSubcoreMesh(core_axis_name="c", subcore_axis_name="s",
                                  num_cores=NC, num_subcores=NS)


def _gather_body(a_hbm, b_hbm, row_hbm, col_hbm, ga_hbm, gb_hbm,
                 idx_r, idx_c, abuf, bbuf,
                 sem_ir, sem_ic, sem_a, sem_b, sem_oa, sem_ob):
    w = lax.axis_index("s") * NC + lax.axis_index("c")

    def bases(t):
        c = w + NW * t
        valid = c < NCHUNK
        return jnp.where(valid, c * CHUNK, E)

    def start_idx(t, slot):
        b = slot * CHUNK
        base = bases(t)
        pltpu.make_async_copy(row_hbm.at[pl.ds(base, CHUNK)],
                              idx_r.at[pl.ds(b, CHUNK)], sem_ir).start()
        pltpu.make_async_copy(col_hbm.at[pl.ds(base, CHUNK)],
                              idx_c.at[pl.ds(b, CHUNK)], sem_ic).start()

    def wait_idx(slot):
        b = slot * CHUNK
        pltpu.make_async_copy(row_hbm.at[pl.ds(0, CHUNK)],
                              idx_r.at[pl.ds(b, CHUNK)], sem_ir).wait()
        pltpu.make_async_copy(col_hbm.at[pl.ds(0, CHUNK)],
                              idx_c.at[pl.ds(b, CHUNK)], sem_ic).wait()

    def wait_out(slot):
        b = slot * CHUNK
        pltpu.make_async_copy(abuf.at[pl.ds(b, CHUNK)],
                              ga_hbm.at[pl.ds(0, CHUNK)], sem_oa).wait()
        pltpu.make_async_copy(bbuf.at[pl.ds(b, CHUNK)],
                              gb_hbm.at[pl.ds(0, CHUNK)], sem_ob).wait()

    def start_gather(js, jg):
        bi = js * CHUNK
        bg = jg * CHUNK
        pltpu.make_async_copy(a_hbm.at[idx_r.at[pl.ds(bi, CHUNK)]],
                              abuf.at[pl.ds(bg, CHUNK)], sem_a).start()
        pltpu.make_async_copy(b_hbm.at[idx_c.at[pl.ds(bi, CHUNK)]],
                              bbuf.at[pl.ds(bg, CHUNK)], sem_b).start()

    def wait_gather(jg):
        bg = jg * CHUNK
        pltpu.make_async_copy(a_hbm.at[pl.ds(0, CHUNK)],
                              abuf.at[pl.ds(bg, CHUNK)], sem_a).wait()
        pltpu.make_async_copy(b_hbm.at[pl.ds(0, CHUNK)],
                              bbuf.at[pl.ds(bg, CHUNK)], sem_b).wait()

    def start_out(t, jg):
        bg = jg * CHUNK
        base = bases(t)
        pltpu.make_async_copy(abuf.at[pl.ds(bg, CHUNK)],
                              ga_hbm.at[pl.ds(base, CHUNK)], sem_oa).start()
        pltpu.make_async_copy(bbuf.at[pl.ds(bg, CHUNK)],
                              gb_hbm.at[pl.ds(base, CHUNK)], sem_ob).start()

    for t in (0, 1, 2, 3, 4):
        start_idx(t, t)
    for t in (0, 1):
        wait_idx(t)
        start_gather(t, t)

    def ringstep(t, js, jg):
        @pl.when(t + 2 < NT)
        def _():
            wait_idx((js + 2) % 6)

            @pl.when(t >= 1)
            def _():
                wait_out((jg + 2) % 3)

            start_gather((js + 2) % 6, (jg + 2) % 3)

        @pl.when(t + 5 < NT)
        def _():
            start_idx(t + 5, (js + 5) % 6)

        wait_gather(jg)
        start_out(t, jg)

    def body(u, carry):
        for j in range(6):
            ringstep(6 * u + j, j, j % 3)
        return carry

    lax.fori_loop(0, (NT - 1) // 6, body, 0)
    ringstep(NT - 1, (NT - 1) % 6, (NT - 1) % 3)
    wait_out((NT - 3) % 3)
    wait_out((NT - 2) % 3)
    wait_out((NT - 1) % 3)


def _sc_gather(a, b, rowp, colp):
    return pl.kernel(
        _gather_body,
        out_type=[jax.ShapeDtypeStruct((E + CHUNK, H), _f32)] * 2,
        mesh=_mesh(),
        scratch_types=[
            pltpu.VMEM((6 * CHUNK,), jnp.int32),
            pltpu.VMEM((6 * CHUNK,), jnp.int32),
            pltpu.VMEM((3 * CHUNK, H), _f32),
            pltpu.VMEM((3 * CHUNK, H), _f32),
            pltpu.SemaphoreType.DMA,
            pltpu.SemaphoreType.DMA,
            pltpu.SemaphoreType.DMA,
            pltpu.SemaphoreType.DMA,
            pltpu.SemaphoreType.DMA,
            pltpu.SemaphoreType.DMA,
        ],
    )(a, b, rowp, colp)


def _scatter_body(eh_hbm, row_hbm, zsum_hbm,
                  sums_o, idx2, rows, sums_sp):
    s = lax.axis_index("s")
    ci = lax.axis_index("c")
    w = s * NC + ci

    for k in range(SZ // CHUNK):
        b = s * SZ + k * CHUNK
        pltpu.sync_copy(zsum_hbm.at[pl.ds(b, CHUNK)], rows)
        pltpu.sync_copy(rows, sums_sp.at[pl.ds(b, CHUNK)])

    plsc.subcore_barrier()

    def step(t, carry):
        c = w + NW * t

        @pl.when(c < NCHUNK)
        def _():
            base = c * CHUNK
            pltpu.sync_copy(row_hbm.at[pl.ds(base, CHUNK)], idx2)
            pltpu.sync_copy(eh_hbm.at[pl.ds(base, CHUNK)], rows)
            pltpu.sync_copy(rows, sums_sp.at[idx2], add=True)

        return carry

    lax.fori_loop(0, NT, step, 0)
    plsc.subcore_barrier()

    for k in range(SZ // CHUNK):
        b = s * SZ + k * CHUNK
        pltpu.sync_copy(sums_sp.at[pl.ds(b, CHUNK)], rows)
        pltpu.sync_copy(rows, sums_o.at[pl.ds(ci * NP + b, CHUNK)])


def _sc_scatter(eh, row, zsum):
    sums = pl.kernel(
        _scatter_body,
        out_type=jax.ShapeDtypeStruct((NC * NP, H), _f32),
        mesh=_mesh(),
        scratch_types=[
            pltpu.VMEM((CHUNK,), jnp.int32),
            pltpu.VMEM((CHUNK, H), _f32),
            pltpu.VMEM_SHARED((NP, H), _f32),
        ],
    )(eh, row, zsum)
    return sums.reshape(NC, NP, H)


def _counts_body(row_hbm, ones_hbm, zsum_hbm, cnt_o, idx2, rows, cnt_sp):
    s = lax.axis_index("s")
    ci = lax.axis_index("c")
    w = s * NC + ci

    for k in range(SZ // CHUNK):
        b = s * SZ + k * CHUNK
        pltpu.sync_copy(zsum_hbm.at[pl.ds(b, CHUNK)], rows)
        pltpu.sync_copy(rows, cnt_sp.at[pl.ds(b, CHUNK)])

    pltpu.sync_copy(ones_hbm, rows)
    plsc.subcore_barrier()

    def step(t, carry):
        c = w + NW * t

        @pl.when(c < NCHUNK)
        def _():
            base = c * CHUNK
            pltpu.sync_copy(row_hbm.at[pl.ds(base, CHUNK)], idx2)
            pltpu.sync_copy(rows, cnt_sp.at[idx2], add=True)

        return carry

    lax.fori_loop(0, NT, step, 0)
    plsc.subcore_barrier()

    for k in range(SZ // CHUNK):
        b = s * SZ + k * CHUNK
        pltpu.sync_copy(cnt_sp.at[pl.ds(b, CHUNK)], rows)
        pltpu.sync_copy(rows, cnt_o.at[pl.ds(ci * NP + b, CHUNK)])


def _sc_counts(row, ones128, zsum):
    cnt = pl.kernel(
        _counts_body,
        out_type=jax.ShapeDtypeStruct((NC * NP, H), _f32),
        mesh=_mesh(),
        scratch_types=[
            pltpu.VMEM((CHUNK,), jnp.int32),
            pltpu.VMEM((CHUNK, H), _f32),
            pltpu.VMEM_SHARED((NP, H), _f32),
        ],
    )(row, ones128, zsum)
    return cnt.reshape(NC, NP, H)


def kernel(x, edge_index, edge_attr, conditions, batch, params):
    row = edge_index[0].astype(jnp.int32)
    col = edge_index[1].astype(jnp.int32)
    batchf = batch.astype(_f32).reshape(N, 1)

    lp = params['layers']
    esp = []
    nsp = []
    for l in range(2):
        ew = lp[l]['edge']
        esp.append({'wa': ew['w1'][0:H], 'wb': ew['w1'][H:2 * H],
                    'wc': ew['w1'][2 * H:3 * H], 'wd': ew['w1'][3 * H:4 * H],
                    'b1': ew['b1'].reshape(1, H), 'w2': ew['w2'],
                    'b2': ew['b2']})
        nw = lp[l]['node']
        nsp.append({'va': nw['w1'][0:H], 'vb': nw['w1'][H:2 * H],
                    'vc': nw['w1'][2 * H:3 * H], 'b1': nw['b1'].reshape(1, H),
                    'w2': nw['w2'], 'b2': nw['b2'].reshape(1, H)})

    uA1, uA2, uC1, uC2 = _prep(conditions, params['cond_enc'],
                               esp[0]['wd'], esp[1]['wd'],
                               nsp[0]['vc'], nsp[1]['vc'])

    xh, oh, a_t, b_t = _k1(x, batchf, params['node_enc'],
                           esp[0]['wa'], esp[0]['wb'], uA1, esp[0]['b1'])
    eh = _k2(edge_attr, params['edge_enc'])

    zsum = jnp.zeros((NP, H), _f32)
    ones128 = jnp.ones((CHUNK, H), _f32)
    cnt = _sc_counts(row, ones128, zsum)
    zpad = jnp.zeros((CHUNK,), jnp.int32)
    rowp = jnp.concatenate([row, zpad])
    colp = jnp.concatenate([col, zpad])

    out = None
    for l in range(2):
        ga, gb = _sc_gather(a_t, b_t, rowp, colp)
        eh = _k3(eh, ga, gb, esp[l]['wc'], esp[l]['w2'], esp[l]['b2'])
        sums = _sc_scatter(eh, row, zsum)
        if l == 0:
            xh, a_t, b_t = _k4a(xh, sums, cnt, oh, nsp[0], uC1,
                                esp[1]['wa'], esp[1]['wb'], uA2, esp[1]['b1'])
        else:
            out = _k4b(xh, sums, cnt, oh, nsp[1], uC2, params['decoder'])
    return out


# speedup vs baseline: 1.2097x; 1.2097x over previous
import jax
import jax.numpy as jnp
from jax import lax
from jax.experimental import pallas as pl
from jax.experimental.pallas import tpu as pltpu
from jax.experimental.pallas import tpu_sc as plsc

N = 10000
E = 320000
B = 16
H = 128

NB_N = 1000
NB_E = 2560
CHUNK = 128
NCHUNK = E // CHUNK
NC, NS = 2, 16
NW = NC * NS
NT = -(-NCHUNK // NW)
NP = 10240
SZ = NP // NS

_f32 = jnp.float32


def _relu(v):
    return jnp.maximum(v, 0.0)


def _prep_body(cond, cw1, cb1, cw2, cb2, wd1, wd2, vc1, vc2,
               uA1, uA2, uC1, uC2):
    u = _relu(jnp.dot(cond[...], cw1[...]) + cb1[...])
    u = jnp.dot(u, cw2[...]) + cb2[...]
    uA1[...] = jnp.dot(u, wd1[...])
    uA2[...] = jnp.dot(u, wd2[...])
    uC1[...] = jnp.dot(u, vc1[...])
    uC2[...] = jnp.dot(u, vc2[...])


def _k1_body(x, bf, w1, b1, w2, b2, wa, wb, ua, b1e,
             xh_o, oh_o, a_o, b_o):
    h = _relu(jnp.dot(x[...], w1[...]) + b1[...])
    xh = jnp.dot(h, w2[...]) + b2[...]
    iot = lax.broadcasted_iota(jnp.int32, (NB_N, B), 1).astype(_f32)
    oh = (bf[...] == iot).astype(_f32)
    xh_o[...] = xh
    oh_o[...] = oh
    a_o[...] = jnp.dot(xh, wa[...]) + jnp.dot(oh, ua[...]) + b1e[...]
    b_o[...] = jnp.dot(xh, wb[...])


def _k2_body(ea, w1, b1, w2, b2, out):
    h = _relu(jnp.dot(ea[...], w1[...]) + b1[...])
    out[...] = jnp.dot(h, w2[...]) + b2[...]


def _k3_body(ehp, ga, gb, w1c, w2, b2, out):
    pre = jnp.dot(ehp[...], w1c[...]) + ga[...] + gb[...]
    out[...] = jnp.dot(_relu(pre), w2[...]) + b2[...]


def _node_update(xh, sums, cnt, oh, va, vb, uc, nb1, nw2, nb2):
    s = sums[0] + sums[1]
    ct = (cnt[0] + cnt[1])[:, 0:1]
    agg = s / jnp.maximum(ct, 1.0)
    h = _relu(jnp.dot(xh, va[...]) + jnp.dot(agg, vb[...])
              + jnp.dot(oh, uc[...]) + nb1[...])
    return jnp.dot(h, nw2[...]) + nb2[...] + xh


def _k4a_body(xh, sums, cnt, oh, va, vb, uc, nb1, nw2, nb2,
              wa2, wb2, ua2, b12, xh2_o, a_o, b_o):
    xh2 = _node_update(xh[...], sums[...], cnt[...], oh[...],
                       va, vb, uc, nb1, nw2, nb2)
    xh2_o[...] = xh2
    a_o[...] = (jnp.dot(xh2, wa2[...]) + jnp.dot(oh[...], ua2[...])
                + b12[...])
    b_o[...] = jnp.dot(xh2, wb2[...])


def _k4b_body(xh, sums, cnt, oh, va, vb, uc, nb1, nw2, nb2,
              dw1, db1, dw2, db2, out):
    xh2 = _node_update(xh[...], sums[...], cnt[...], oh[...],
                       va, vb, uc, nb1, nw2, nb2)
    h = _relu(jnp.dot(xh2, dw1[...]) + db1[...])
    out[...] = jnp.dot(h, dw2[...]) + db2[...]


def _full(shape):
    return pl.BlockSpec(shape, lambda *_: tuple(0 for _ in shape))


def _prep(cond, ce, wd1, wd2, vc1, vc2):
    outs = [jax.ShapeDtypeStruct((B, H), _f32)] * 4
    return pl.pallas_call(
        _prep_body,
        out_shape=outs,
    )(cond, ce['w1'], ce['b1'].reshape(1, H), ce['w2'], ce['b2'].reshape(1, H),
      wd1, wd2, vc1, vc2)


def _k1(x, batchf, ne, wa, wb, ua, b1e):
    nb = N // NB_N
    row_spec = pl.BlockSpec((NB_N, H), lambda i: (i, 0))
    outs = [jax.ShapeDtypeStruct((N, H), _f32),
            jax.ShapeDtypeStruct((N, B), _f32),
            jax.ShapeDtypeStruct((N, H), _f32),
            jax.ShapeDtypeStruct((N, H), _f32)]
    return pl.pallas_call(
        _k1_body,
        grid=(nb,),
        in_specs=[row_spec,
                  pl.BlockSpec((NB_N, 1), lambda i: (i, 0)),
                  _full((H, H)), _full((1, H)), _full((H, H)), _full((1, H)),
                  _full((H, H)), _full((H, H)), _full((B, H)), _full((1, H))],
        out_specs=[row_spec,
                   pl.BlockSpec((NB_N, B), lambda i: (i, 0)),
                   row_spec, row_spec],
        out_shape=outs,
    )(x, batchf, ne['w1'], ne['b1'].reshape(1, H), ne['w2'],
      ne['b2'].reshape(1, H), wa, wb, ua, b1e)


def _k2(ea, ee):
    nb = E // NB_E
    return pl.pallas_call(
        _k2_body,
        grid=(nb,),
        in_specs=[pl.BlockSpec((NB_E, 16), lambda i: (i, 0)),
                  _full((16, H)), _full((1, H)), _full((H, H)), _full((1, H))],
        out_specs=pl.BlockSpec((NB_E, H), lambda i: (i, 0)),
        out_shape=jax.ShapeDtypeStruct((E, H), _f32),
    )(ea, ee['w1'], ee['b1'].reshape(1, H), ee['w2'], ee['b2'].reshape(1, H))


def _k3(ehp, ga, gb, w1c, w2, b2):
    nb = E // NB_E
    row_spec = pl.BlockSpec((NB_E, H), lambda i: (i, 0))
    return pl.pallas_call(
        _k3_body,
        grid=(nb,),
        in_specs=[row_spec, row_spec, row_spec,
                  _full((H, H)), _full((H, H)), _full((1, H))],
        out_specs=row_spec,
        out_shape=jax.ShapeDtypeStruct((E, H), _f32),
    )(ehp, ga, gb, w1c, w2, b2.reshape(1, H))


def _k4a(xh, sums, cnt, oh, nl, uc, wa2, wb2, ua2, b12):
    nb = N // NB_N
    row_spec = pl.BlockSpec((NB_N, H), lambda i: (i, 0))
    outs = [jax.ShapeDtypeStruct((N, H), _f32)] * 3
    return pl.pallas_call(
        _k4a_body,
        grid=(nb,),
        in_specs=[row_spec,
                  pl.BlockSpec((NC, NB_N, H), lambda i: (0, i, 0)),
                  pl.BlockSpec((NC, NB_N, H), lambda i: (0, i, 0)),
                  pl.BlockSpec((NB_N, B), lambda i: (i, 0)),
                  _full((H, H)), _full((H, H)), _full((B, H)), _full((1, H)),
                  _full((H, H)), _full((1, H)),
                  _full((H, H)), _full((H, H)), _full((B, H)), _full((1, H))],
        out_specs=[row_spec, row_spec, row_spec],
        out_shape=outs,
    )(xh, sums, cnt, oh,
      nl['va'], nl['vb'], uc, nl['b1'], nl['w2'], nl['b2'],
      wa2, wb2, ua2, b12)


def _k4b(xh, sums, cnt, oh, nl, uc, dec):
    nb = N // NB_N
    row_spec = pl.BlockSpec((NB_N, H), lambda i: (i, 0))
    return pl.pallas_call(
        _k4b_body,
        grid=(nb,),
        in_specs=[row_spec,
                  pl.BlockSpec((NC, NB_N, H), lambda i: (0, i, 0)),
                  pl.BlockSpec((NC, NB_N, H), lambda i: (0, i, 0)),
                  pl.BlockSpec((NB_N, B), lambda i: (i, 0)),
                  _full((H, H)), _full((H, H)), _full((B, H)), _full((1, H)),
                  _full((H, H)), _full((1, H)),
                  _full((H, H)), _full((1, H)), _full((H, H)), _full((1, H))],
        out_specs=row_spec,
        out_shape=jax.ShapeDtypeStruct((N, H), _f32),
    )(xh, sums, cnt, oh,
      nl['va'], nl['vb'], uc, nl['b1'], nl['w2'], nl['b2'],
      dec['w1'], dec['b1'].reshape(1, H), dec['w2'], dec['b2'].reshape(1, H))


def _mesh():
    return plsc.VectorSubcoreMesh(core_axis_name="c", subcore_axis_name="s",
                                  num_cores=NC, num_subcores=NS)


def _gather_body(a_hbm, b_hbm, row_hbm, col_hbm, ga_hbm, gb_hbm,
                 idx_r, idx_c, abuf, bbuf, sem_a, sem_b):
    w = lax.axis_index("s") * NC + lax.axis_index("c")

    def step(t, carry):
        c = w + NW * t

        @pl.when(c < NCHUNK)
        def _():
            base = c * CHUNK
            pltpu.sync_copy(row_hbm.at[pl.ds(base, CHUNK)], idx_r)
            pltpu.sync_copy(col_hbm.at[pl.ds(base, CHUNK)], idx_c)
            ca = pltpu.async_copy(a_hbm.at[idx_r], abuf, sem_a)
            cb = pltpu.async_copy(b_hbm.at[idx_c], bbuf, sem_b)
            ca.wait()
            cb.wait()
            pltpu.sync_copy(abuf, ga_hbm.at[pl.ds(base, CHUNK)])
            pltpu.sync_copy(bbuf, gb_hbm.at[pl.ds(base, CHUNK)])

        return carry

    lax.fori_loop(0, NT, step, 0)


def _sc_gather(a, b, row, col):
    return pl.kernel(
        _gather_body,
        out_type=[jax.ShapeDtypeStruct((E, H), _f32)] * 2,
        mesh=_mesh(),
        scratch_types=[
            pltpu.VMEM((CHUNK,), jnp.int32),
            pltpu.VMEM((CHUNK,), jnp.int32),
            pltpu.VMEM((CHUNK, H), _f32),
            pltpu.VMEM((CHUNK, H), _f32),
            pltpu.SemaphoreType.DMA,
            pltpu.SemaphoreType.DMA,
        ],
    )(a, b, row, col)


def _scatter_body(eh_hbm, rowp_hbm, zsum_hbm,
                  sums_o, idx2, rows, sums_sp, sem_i, sem_r, sem_d0, sem_d1):
    s = lax.axis_index("s")
    ci = lax.axis_index("c")
    w = s * NC + ci

    for k in range(SZ // CHUNK):
        b = s * SZ + k * CHUNK
        pltpu.sync_copy(zsum_hbm.at[pl.ds(b, CHUNK)],
                        rows.at[pl.ds(0, CHUNK)])
        pltpu.sync_copy(rows.at[pl.ds(0, CHUNK)], sums_sp.at[pl.ds(b, CHUNK)])

    plsc.subcore_barrier()

    def start_loads(t, sl):
        c = w + NW * t
        valid = c < NCHUNK
        ib = jnp.where(valid, c * CHUNK, E)
        eb = jnp.where(valid, c * CHUNK, 0)
        pltpu.make_async_copy(rowp_hbm.at[pl.ds(ib, CHUNK)],
                              idx2.at[sl], sem_i).start()
        pltpu.make_async_copy(eh_hbm.at[pl.ds(eb, CHUNK)],
                              rows.at[pl.ds(sl * CHUNK, CHUNK)], sem_r).start()

    def wait_loads(sl):
        pltpu.make_async_copy(rowp_hbm.at[pl.ds(0, CHUNK)],
                              idx2.at[sl], sem_i).wait()
        pltpu.make_async_copy(eh_hbm.at[pl.ds(0, CHUNK)],
                              rows.at[pl.ds(sl * CHUNK, CHUNK)], sem_r).wait()

    def start_add(sl):
        sem = sem_d0 if sl == 0 else sem_d1
        pltpu.make_async_copy(rows.at[pl.ds(sl * CHUNK, CHUNK)],
                              sums_sp.at[idx2.at[sl]], sem).start()

    def wait_add(sl):
        sem = sem_d0 if sl == 0 else sem_d1
        pltpu.make_async_copy(rows.at[pl.ds(sl * CHUNK, CHUNK)],
                              sums_sp.at[pl.ds(0, CHUNK)], sem).wait()

    start_loads(0, 0)

    def pstep(t, sl):
        wait_loads(sl)

        @pl.when(t >= 1)
        def _():
            wait_add(1 - sl)

        @pl.when(t + 1 < NT)
        def _():
            start_loads(t + 1, 1 - sl)

        start_add(sl)

    def body(u, carry):
        pstep(2 * u, 0)
        pstep(2 * u + 1, 1)
        return carry

    lax.fori_loop(0, (NT - 1) // 2, body, 0)
    pstep(NT - 1, (NT - 1) % 2)
    wait_add((NT - 1) % 2)
    plsc.subcore_barrier()

    for k in range(SZ // CHUNK):
        b = s * SZ + k * CHUNK
        pltpu.sync_copy(sums_sp.at[pl.ds(b, CHUNK)],
                        rows.at[pl.ds(0, CHUNK)])
        pltpu.sync_copy(rows.at[pl.ds(0, CHUNK)],
                        sums_o.at[pl.ds(ci * NP + b, CHUNK)])


def _sc_scatter(eh, rowp, zsum):
    sums = pl.kernel(
        _scatter_body,
        out_type=jax.ShapeDtypeStruct((NC * NP, H), _f32),
        mesh=_mesh(),
        scratch_types=[
            pltpu.VMEM((2, CHUNK), jnp.int32),
            pltpu.VMEM((2 * CHUNK, H), _f32),
            pltpu.VMEM_SHARED((NP, H), _f32),
            pltpu.SemaphoreType.DMA,
            pltpu.SemaphoreType.DMA,
            pltpu.SemaphoreType.DMA,
            pltpu.SemaphoreType.DMA,
        ],
    )(eh, rowp, zsum)
    return sums.reshape(NC, NP, H)


def _counts_body(row_hbm, ones_hbm, zsum_hbm, cnt_o, idx2, rows, cnt_sp):
    s = lax.axis_index("s")
    ci = lax.axis_index("c")
    w = s * NC + ci

    for k in range(SZ // CHUNK):
        b = s * SZ + k * CHUNK
        pltpu.sync_copy(zsum_hbm.at[pl.ds(b, CHUNK)], rows)
        pltpu.sync_copy(rows, cnt_sp.at[pl.ds(b, CHUNK)])

    pltpu.sync_copy(ones_hbm, rows)
    plsc.subcore_barrier()

    def step(t, carry):
        c = w + NW * t

        @pl.when(c < NCHUNK)
        def _():
            base = c * CHUNK
            pltpu.sync_copy(row_hbm.at[pl.ds(base, CHUNK)], idx2)
            pltpu.sync_copy(rows, cnt_sp.at[idx2], add=True)

        return carry

    lax.fori_loop(0, NT, step, 0)
    plsc.subcore_barrier()

    for k in range(SZ // CHUNK):
        b = s * SZ + k * CHUNK
        pltpu.sync_copy(cnt_sp.at[pl.ds(b, CHUNK)], rows)
        pltpu.sync_copy(rows, cnt_o.at[pl.ds(ci * NP + b, CHUNK)])


def _sc_counts(row, ones128, zsum):
    cnt = pl.kernel(
        _counts_body,
        out_type=jax.ShapeDtypeStruct((NC * NP, H), _f32),
        mesh=_mesh(),
        scratch_types=[
            pltpu.VMEM((CHUNK,), jnp.int32),
            pltpu.VMEM((CHUNK, H), _f32),
            pltpu.VMEM_SHARED((NP, H), _f32),
        ],
    )(row, ones128, zsum)
    return cnt.reshape(NC, NP, H)


def kernel(x, edge_index, edge_attr, conditions, batch, params):
    row = edge_index[0].astype(jnp.int32)
    col = edge_index[1].astype(jnp.int32)
    batchf = batch.astype(_f32).reshape(N, 1)

    lp = params['layers']
    esp = []
    nsp = []
    for l in range(2):
        ew = lp[l]['edge']
        esp.append({'wa': ew['w1'][0:H], 'wb': ew['w1'][H:2 * H],
                    'wc': ew['w1'][2 * H:3 * H], 'wd': ew['w1'][3 * H:4 * H],
                    'b1': ew['b1'].reshape(1, H), 'w2': ew['w2'],
                    'b2': ew['b2']})
        nw = lp[l]['node']
        nsp.append({'va': nw['w1'][0:H], 'vb': nw['w1'][H:2 * H],
                    'vc': nw['w1'][2 * H:3 * H], 'b1': nw['b1'].reshape(1, H),
                    'w2': nw['w2'], 'b2': nw['b2'].reshape(1, H)})

    uA1, uA2, uC1, uC2 = _prep(conditions, params['cond_enc'],
                               esp[0]['wd'], esp[1]['wd'],
                               nsp[0]['vc'], nsp[1]['vc'])

    xh, oh, a_t, b_t = _k1(x, batchf, params['node_enc'],
                           esp[0]['wa'], esp[0]['wb'], uA1, esp[0]['b1'])
    eh = _k2(edge_attr, params['edge_enc'])

    zsum = jnp.zeros((NP, H), _f32)
    ones128 = jnp.ones((CHUNK, H), _f32)
    cnt = _sc_counts(row, ones128, zsum)
    zpad = jnp.zeros((CHUNK,), jnp.int32)
    rowp = jnp.concatenate([row, zpad])
    colp = jnp.concatenate([col, zpad])
    rows_pad = jnp.concatenate([row, jnp.full((CHUNK,), N, jnp.int32)])

    out = None
    for l in range(2):
        ga, gb = _sc_gather(a_t, b_t, rowp, colp)
        eh = _k3(eh, ga, gb, esp[l]['wc'], esp[l]['w2'], esp[l]['b2'])
        sums = _sc_scatter(eh, rows_pad, zsum)
        if l == 0:
            xh, a_t, b_t = _k4a(xh, sums, cnt, oh, nsp[0], uC1,
                                esp[1]['wa'], esp[1]['wb'], uA2, esp[1]['b1'])
        else:
            out = _k4b(xh, sums, cnt, oh, nsp[1], uC2, params['decoder'])
    return out
